# baseline (speedup 1.0000x reference)
import functools

import jax
import jax.numpy as jnp
from jax import lax
from jax.experimental import pallas as pl
from jax.experimental.pallas import tpu as pltpu
from jax.experimental.pallas import tpu_sc as plsc

B = 16384
H = 128
HW = H // 2
NC, NS = 2, 16
NW = NC * NS
CB = B // 2
BPW = CB // NW
NCH = BPW // H
BB = 2048

_MESH = plsc.VectorSubcoreMesh(core_axis_name="c", subcore_axis_name="s")


def _sc_gather_body(ip_hbm, in_hbm, tp_hbm, tn_hbm, outp_hbm, outn_hbm,
                    idxp_v, idxn_v, rows_p, rows_n, sem_g, sem_w):
    wid = lax.axis_index("s") * NC + lax.axis_index("c")
    base = wid * BPW
    pltpu.sync_copy(ip_hbm.at[wid], idxp_v)
    pltpu.sync_copy(in_hbm.at[wid], idxn_v)
    gp = [pltpu.make_async_copy(tp_hbm.at[idxp_v.at[j]],
                                rows_p.at[pl.ds(j * H, H)], sem_g)
          for j in range(NCH)]
    gn = [pltpu.make_async_copy(tn_hbm.at[idxn_v.at[j]],
                                rows_n.at[pl.ds(j * H, H)], sem_g)
          for j in range(NCH)]
    for c in gp + gn:
        c.start()
    for c in gp:
        c.wait()
    wp = pltpu.make_async_copy(rows_p, outp_hbm.at[pl.ds(base, BPW)], sem_w)
    wp.start()
    for c in gn:
        c.wait()
    wn = pltpu.make_async_copy(rows_n, outn_hbm.at[pl.ds(base, BPW)], sem_w)
    wn.start()
    wp.wait()
    wn.wait()


def _sc_gather_f32(idx3_p, idx3_n, emb_p, emb_n):
    k = functools.partial(
        pl.kernel,
        out_type=(jax.ShapeDtypeStruct((CB, H), jnp.float32),
                  jax.ShapeDtypeStruct((CB, H), jnp.float32)),
        mesh=_MESH,
        scratch_types=[
            pltpu.VMEM((NCH, H), jnp.int32),
            pltpu.VMEM((NCH, H), jnp.int32),
            pltpu.VMEM((BPW, H), jnp.float32),
            pltpu.VMEM((BPW, H), jnp.float32),
            pltpu.SemaphoreType.DMA,
            pltpu.SemaphoreType.DMA,
        ],
    )(_sc_gather_body)
    return k(idx3_p, idx3_n, emb_p, emb_n)


def _sc_gather_packed(idx3_p, idx3_n, tbl_p, tbl_n):
    k = functools.partial(
        pl.kernel,
        out_type=(jax.ShapeDtypeStruct((CB, HW), jnp.int32),
                  jax.ShapeDtypeStruct((CB, HW), jnp.int32)),
        mesh=_MESH,
        compiler_params=pltpu.CompilerParams(use_tc_tiling_on_sc=False),
        scratch_types=[
            pltpu.VMEM((NCH, H), jnp.int32),
            pltpu.VMEM((NCH, H), jnp.int32),
            pltpu.VMEM((BPW, HW), jnp.int32),
            pltpu.VMEM((BPW, HW), jnp.int32),
            pltpu.SemaphoreType.DMA,
            pltpu.SemaphoreType.DMA,
        ],
    )(_sc_gather_body)
    return k(idx3_p, idx3_n, tbl_p, tbl_n)


def _norm_scale(f):
    return lax.rsqrt(jnp.maximum(
        jnp.sum(f * f, axis=1, keepdims=True), 1e-24))


def _mlp_tail(h, w2_ref, b2_ref, w3_ref):
    h = jnp.dot(h, w2_ref[...], preferred_element_type=jnp.float32)
    h = jnp.maximum(h + b2_ref[...], 0.0)
    return lax.dot_general(w3_ref[...], h, (((1,), (1,)), ((), ())),
                           preferred_element_type=jnp.float32)


_CONST = lambda i: (0, 0)
_W_SPECS = [
    pl.BlockSpec((H, H), _CONST),
    pl.BlockSpec((H, H), _CONST),
    pl.BlockSpec((1, H), _CONST),
    pl.BlockSpec((H, H), _CONST),
    pl.BlockSpec((1, H), _CONST),
    pl.BlockSpec((1, H), _CONST),
    pl.BlockSpec((1, 1), _CONST),
]


def _tc_mlp_f32(gp, gn, w1p, w1n, b1, w2, b2, w3r, b3):
    def body(gp_ref, gn_ref, w1p_ref, w1n_ref, b1_ref, w2_ref, b2_ref,
             w3_ref, b3_ref, out_ref):
        p = gp_ref[...]
        n = gn_ref[...]
        h = _norm_scale(p) * jnp.dot(
            p, w1p_ref[...], preferred_element_type=jnp.float32)
        h = h + _norm_scale(n) * jnp.dot(
            n, w1n_ref[...], preferred_element_type=jnp.float32)
        h = jnp.maximum(h + b1_ref[...], 0.0)
        out_ref[...] = _mlp_tail(h, w2_ref, b2_ref, w3_ref) + b3_ref[...]

    return pl.pallas_call(
        body,
        grid=(CB // BB,),
        in_specs=[pl.BlockSpec((BB, H), lambda i: (i, 0)),
                  pl.BlockSpec((BB, H), lambda i: (i, 0))] + _W_SPECS,
        out_specs=pl.BlockSpec((1, BB), lambda i: (0, i)),
        out_shape=jax.ShapeDtypeStruct((1, CB), jnp.float32),
    )(gp, gn, w1p, w1n, b1, w2, b2, w3r, b3)


def _unpack(raw):
    lo = lax.bitcast_convert_type(lax.shift_left(raw, 16), jnp.float32)
    hi = lax.bitcast_convert_type(
        lax.bitwise_and(raw, jnp.int32(-65536)), jnp.float32)
    return lo, hi


def _feats(lo, hi, s):
    c = slice(s * HW, s * HW + HW)
    return jnp.concatenate([lo[:, c], hi[:, c]], axis=1)


def _tc_mlp_packed(gp, gn, w1p, w1n, b1, w2, b2, w3r, b3):
    R = BB // 2

    def body(gp_ref, gn_ref, w1p_ref, w1n_ref, b1_ref, w2_ref, b2_ref,
             w3_ref, b3_ref, out_ref):
        plo, phi = _unpack(gp_ref[...])
        nlo, nhi = _unpack(gn_ref[...])
        outs = []
        for s in (0, 1):
            p = _feats(plo, phi, s)
            n = _feats(nlo, nhi, s)
            h = _norm_scale(p) * jnp.dot(
                p, w1p_ref[...], preferred_element_type=jnp.float32)
            h = h + _norm_scale(n) * jnp.dot(
                n, w1n_ref[...], preferred_element_type=jnp.float32)
            h = jnp.maximum(h + b1_ref[...], 0.0)
            outs.append(_mlp_tail(h, w2_ref, b2_ref, w3_ref))
        oa, ob = outs
        segs = []
        for w in range(R // H):
            segs.append(oa[:, w * H:(w + 1) * H])
            segs.append(ob[:, w * H:(w + 1) * H])
        out_ref[...] = jnp.concatenate(segs, axis=1) + b3_ref[...]

    return pl.pallas_call(
        body,
        grid=(CB // BB,),
        in_specs=[pl.BlockSpec((R, H), lambda i: (i, 0)),
                  pl.BlockSpec((R, H), lambda i: (i, 0))] + _W_SPECS,
        out_specs=pl.BlockSpec((1, BB), lambda i: (0, i)),
        out_shape=jax.ShapeDtypeStruct((1, CB), jnp.float32),
    )(gp, gn, w1p, w1n, b1, w2, b2, w3r, b3)


def _pack_table(t):
    t16 = t.astype(jnp.bfloat16)
    lo = lax.bitcast_convert_type(t16[:, :HW], jnp.uint16).astype(jnp.uint32)
    hi = lax.bitcast_convert_type(t16[:, HW:], jnp.uint16).astype(jnp.uint32)
    return lax.bitcast_convert_type((hi << 16) | lo, jnp.int32)


def _perm_idx(col):
    i3 = col.reshape(NW, 2, BPW // 2)
    return jnp.swapaxes(i3, 1, 2).reshape(NW, NCH, H)


def kernel(x, emb_proton, emb_neutron, W1, b1, W2, b2, W3, b3):
    x = x.astype(jnp.int32)
    x1, x2 = x[:CB], x[CB:]
    w1p, w1n = W1[:H], W1[H:]
    b1r, b2r = b1.reshape(1, H), b2.reshape(1, H)
    w3r, b3r = W3.reshape(1, H), b3.reshape(1, 1)
    gp1, gn1 = _sc_gather_f32(x1[:, 0].reshape(NW, NCH, H),
                              x1[:, 1].reshape(NW, NCH, H),
                              emb_proton, emb_neutron)
    tbl_p, tbl_n = _pack_table(emb_proton), _pack_table(emb_neutron)
    gp2, gn2 = _sc_gather_packed(_perm_idx(x2[:, 0]), _perm_idx(x2[:, 1]),
                                 tbl_p, tbl_n)
    o1 = _tc_mlp_f32(gp1, gn1, w1p, w1n, b1r, W2, b2r, w3r, b3r)
    o2 = _tc_mlp_packed(gp2.reshape(CB // 2, H), gn2.reshape(CB // 2, H),
                        w1p, w1n, b1r, W2, b2r, w3r, b3r)
    return jnp.concatenate([o1, o2], axis=1).reshape(B, 1)


# speedup vs baseline: 1.0810x; 1.0810x over previous
import functools

import jax
import jax.numpy as jnp
from jax import lax
from jax.experimental import pallas as pl
from jax.experimental.pallas import tpu as pltpu
from jax.experimental.pallas import tpu_sc as plsc

B = 16384
H = 128
HW = H // 2
NC, NS = 2, 16
NW = NC * NS
NCHUNK = 2
CB = B // NCHUNK
BPW = CB // NW
NCH = BPW // H
BB = 2048


def _sc_gather(idx3_p, idx3_n, tbl_p, tbl_n):
    mesh = plsc.VectorSubcoreMesh(core_axis_name="c", subcore_axis_name="s")

    @functools.partial(
        pl.kernel,
        out_type=(
            jax.ShapeDtypeStruct((CB, HW), jnp.int32),
            jax.ShapeDtypeStruct((CB, HW), jnp.int32),
        ),
        mesh=mesh,
        compiler_params=pltpu.CompilerParams(use_tc_tiling_on_sc=False),
        scratch_types=[
            pltpu.VMEM((NCH, H), jnp.int32),
            pltpu.VMEM((NCH, H), jnp.int32),
            pltpu.VMEM((BPW, HW), jnp.int32),
            pltpu.VMEM((BPW, HW), jnp.int32),
            pltpu.SemaphoreType.DMA,
            pltpu.SemaphoreType.DMA,
        ],
    )
    def k(ip_hbm, in_hbm, tp_hbm, tn_hbm, outp_hbm, outn_hbm,
          idxp_v, idxn_v, rows_p, rows_n, sem_g, sem_w):
        wid = lax.axis_index("s") * NC + lax.axis_index("c")
        base = wid * BPW
        pltpu.sync_copy(ip_hbm.at[wid], idxp_v)
        pltpu.sync_copy(in_hbm.at[wid], idxn_v)
        gp = [pltpu.make_async_copy(tp_hbm.at[idxp_v.at[j]],
                                    rows_p.at[pl.ds(j * H, H)], sem_g)
              for j in range(NCH)]
        gn = [pltpu.make_async_copy(tn_hbm.at[idxn_v.at[j]],
                                    rows_n.at[pl.ds(j * H, H)], sem_g)
              for j in range(NCH)]
        for c in gp + gn:
            c.start()
        for c in gp:
            c.wait()
        wp = pltpu.make_async_copy(rows_p, outp_hbm.at[pl.ds(base, BPW)],
                                   sem_w)
        wp.start()
        for c in gn:
            c.wait()
        wn = pltpu.make_async_copy(rows_n, outn_hbm.at[pl.ds(base, BPW)],
                                   sem_w)
        wn.start()
        wp.wait()
        wn.wait()

    return k(idx3_p, idx3_n, tbl_p, tbl_n)


def _unpack(raw):
    lo = lax.bitcast_convert_type(lax.shift_left(raw, 16), jnp.float32)
    hi = lax.bitcast_convert_type(
        lax.bitwise_and(raw, jnp.int32(-65536)), jnp.float32)
    return lo, hi


def _feats(lo, hi, s):
    c = slice(s * HW, s * HW + HW)
    return jnp.concatenate([lo[:, c], hi[:, c]], axis=1)


def _norm_scale(f):
    return lax.rsqrt(jnp.maximum(
        jnp.sum(f * f, axis=1, keepdims=True), 1e-24))


def _tc_mlp(gp, gn, w1p, w1n, b1, w2, b2, w3r, b3):
    R = BB // 2
    grid = (CB // BB,)

    def body(gp_ref, gn_ref, w1p_ref, w1n_ref, b1_ref, w2_ref, b2_ref,
             w3_ref, b3_ref, out_ref):
        plo, phi = _unpack(gp_ref[...])
        nlo, nhi = _unpack(gn_ref[...])
        outs = []
        for s in (0, 1):
            p = _feats(plo, phi, s)
            n = _feats(nlo, nhi, s)
            h = _norm_scale(p) * jnp.dot(
                p, w1p_ref[...], preferred_element_type=jnp.float32)
            h = h + _norm_scale(n) * jnp.dot(
                n, w1n_ref[...], preferred_element_type=jnp.float32)
            h = jnp.maximum(h + b1_ref[...], 0.0)
            h = jnp.dot(h, w2_ref[...], preferred_element_type=jnp.float32)
            h = jnp.maximum(h + b2_ref[...], 0.0)
            outs.append(lax.dot_general(
                w3_ref[...], h, (((1,), (1,)), ((), ())),
                preferred_element_type=jnp.float32))
        oa, ob = outs
        segs = []
        for w in range(R // H):
            segs.append(oa[:, w * H:(w + 1) * H])
            segs.append(ob[:, w * H:(w + 1) * H])
        out_ref[...] = jnp.concatenate(segs, axis=1) + b3_ref[...]

    const = lambda i: (0, 0)
    return pl.pallas_call(
        body,
        grid=grid,
        in_specs=[
            pl.BlockSpec((R, H), lambda i: (i, 0)),
            pl.BlockSpec((R, H), lambda i: (i, 0)),
            pl.BlockSpec((H, H), const),
            pl.BlockSpec((H, H), const),
            pl.BlockSpec((1, H), const),
            pl.BlockSpec((H, H), const),
            pl.BlockSpec((1, H), const),
            pl.BlockSpec((1, H), const),
            pl.BlockSpec((1, 1), const),
        ],
        out_specs=pl.BlockSpec((1, BB), lambda i: (0, i)),
        out_shape=jax.ShapeDtypeStruct((1, CB), jnp.float32),
    )(gp, gn, w1p, w1n, b1, w2, b2, w3r, b3)


def _pack_table(t):
    t16 = t.astype(jnp.bfloat16)
    lo = lax.bitcast_convert_type(t16[:, :HW], jnp.uint16).astype(jnp.uint32)
    hi = lax.bitcast_convert_type(t16[:, HW:], jnp.uint16).astype(jnp.uint32)
    return lax.bitcast_convert_type((hi << 16) | lo, jnp.int32)


def _perm_idx(col):
    i3 = col.reshape(NW, 2, BPW // 2)
    return jnp.swapaxes(i3, 1, 2).reshape(NW, NCH, H)


def kernel(x, emb_proton, emb_neutron, W1, b1, W2, b2, W3, b3):
    x = x.astype(jnp.int32)
    tbl_p = _pack_table(emb_proton)
    tbl_n = _pack_table(emb_neutron)
    w1p, w1n = W1[:H], W1[H:]
    b1r, b2r = b1.reshape(1, H), b2.reshape(1, H)
    w3r, b3r = W3.reshape(1, H), b3.reshape(1, 1)
    outs = []
    for c in range(NCHUNK):
        xc = x[c * CB:(c + 1) * CB]
        gp, gn = _sc_gather(_perm_idx(xc[:, 0]), _perm_idx(xc[:, 1]),
                            tbl_p, tbl_n)
        outs.append(_tc_mlp(gp.reshape(CB // 2, H), gn.reshape(CB // 2, H),
                            w1p, w1n, b1r, W2, b2r, w3r, b3r))
    return jnp.concatenate(outs, axis=1).reshape(B, 1)
